# baseline (speedup 1.0000x reference)
import math
import functools

import jax
import jax.numpy as jnp
from jax import lax
from jax.experimental import pallas as pl
from jax.experimental.pallas import tpu as pltpu
from jax.experimental.pallas import tpu_sc as plsc

_NC = 2
_NS = 16
_NW = _NC * _NS
_L = 16


def _sc_lookup(coords_flat, row_embed, col_embed, *, grid_size, num_emb, dh):
  total = coords_flat.shape[0]
  points = total // 2
  ppw = points // _NW
  cpw = 2 * ppw
  chunk = 64
  n_chunks = ppw // chunk
  mchunk = 8192
  n_max_chunks = total // mchunk

  mesh = plsc.VectorSubcoreMesh(
      core_axis_name="c", subcore_axis_name="s",
      num_cores=_NC, num_subcores=_NS)

  @functools.partial(
      pl.kernel,
      out_type=jax.ShapeDtypeStruct((points, 2 * dh), jnp.float32),
      mesh=mesh,
      compiler_params=pltpu.CompilerParams(needs_layout_passes=False),
      scratch_types=[
          pltpu.VMEM((mchunk,), jnp.float32),
          pltpu.VMEM((cpw,), jnp.float32),
          pltpu.VMEM((ppw,), jnp.int32),
          pltpu.VMEM((ppw,), jnp.int32),
          pltpu.VMEM((chunk, 2 * dh), jnp.float32),
          pltpu.VMEM((chunk, 2 * dh), jnp.float32),
          pltpu.SemaphoreType.DMA,
          pltpu.SemaphoreType.DMA,
          pltpu.SemaphoreType.DMA,
          pltpu.SemaphoreType.DMA,
      ],
  )
  def body(coords_hbm, row_hbm, col_hbm, out_hbm,
           mbuf, cbuf, irow, icol, kb0, kb1,
           gsem0, gsem1, wsem0, wsem1):
    wid = lax.axis_index("s") * _NC + lax.axis_index("c")
    kbufs = (kb0, kb1)
    gsems = (gsem0, gsem1)
    wsems = (wsem0, wsem1)

    def max_step(j, accs):
      pltpu.sync_copy(coords_hbm.at[pl.ds(j * mchunk, mchunk)], mbuf)
      def red(i, accs):
        a0, a1, a2, a3 = accs
        base = i * (4 * _L)
        a0 = jnp.maximum(a0, mbuf[pl.ds(base, _L)])
        a1 = jnp.maximum(a1, mbuf[pl.ds(base + _L, _L)])
        a2 = jnp.maximum(a2, mbuf[pl.ds(base + 2 * _L, _L)])
        a3 = jnp.maximum(a3, mbuf[pl.ds(base + 3 * _L, _L)])
        return (a0, a1, a2, a3)
      return lax.fori_loop(0, mchunk // (4 * _L), red, accs)

    neg = jnp.full((_L,), -jnp.inf, dtype=jnp.float32)
    a0, a1, a2, a3 = lax.fori_loop(0, n_max_chunks, max_step,
                                   (neg, neg, neg, neg))
    acc = jnp.maximum(jnp.maximum(a0, a1), jnp.maximum(a2, a3))
    iota = lax.iota(jnp.int32, _L)
    for s in (1, 2, 4, 8):
      acc = jnp.maximum(acc, acc.at[iota ^ s].get(mode="promise_in_bounds"))
    max_coord = acc

    pltpu.sync_copy(coords_hbm.at[pl.ds(wid * cpw, cpw)], cbuf)
    gs = jnp.float32(grid_size)

    def idx_step(g, _):
      base = g * (2 * _L)
      xi = base + 2 * iota
      x = plsc.load_gather(cbuf, [xi])
      y = plsc.load_gather(cbuf, [xi + 1])
      r = jnp.clip((y / max_coord * gs).astype(jnp.int32), 0, num_emb - 1)
      c = jnp.clip((x / max_coord * gs).astype(jnp.int32), 0, num_emb - 1)
      irow[pl.ds(g * _L, _L)] = r
      icol[pl.ds(g * _L, _L)] = c
      return 0

    lax.fori_loop(0, ppw // _L, idx_step, 0)

    def g_copies(k, b):
      p0 = k * chunk
      return (
          pltpu.make_async_copy(
              row_hbm.at[irow.at[pl.ds(p0, chunk)]],
              kbufs[b].at[:, pl.ds(0, dh)], gsems[b]),
          pltpu.make_async_copy(
              col_hbm.at[icol.at[pl.ds(p0, chunk)]],
              kbufs[b].at[:, pl.ds(dh, dh)], gsems[b]),
      )

    def w_copies(k, b):
      o0 = wid * ppw + k * chunk
      return (
          pltpu.make_async_copy(
              kbufs[b], out_hbm.at[pl.ds(o0, chunk)], wsems[b]),
      )

    def issue(copies):
      for c in copies:
        c.start()

    def drain(copies):
      for c in copies:
        c.wait()

    issue(g_copies(0, 0))

    def pipe_step(j, _):
      k0 = 2 * j
      k1 = k0 + 1

      @pl.when(j > 0)
      def _():
        drain(w_copies(k1 - 2, 1))
      issue(g_copies(k1, 1))
      drain(g_copies(k0, 0))
      issue(w_copies(k0, 0))

      @pl.when(j < n_chunks // 2 - 1)
      def _():
        drain(w_copies(k0, 0))
        issue(g_copies(k0 + 2, 0))
      drain(g_copies(k1, 1))
      issue(w_copies(k1, 1))
      return 0

    lax.fori_loop(0, n_chunks // 2, pipe_step, 0)
    drain(w_copies(n_chunks - 2, 0))
    drain(w_copies(n_chunks - 1, 1))

  return body(coords_flat, row_embed, col_embed)


def kernel(patch_coords, row_embed, col_embed):
  b, n, _ = patch_coords.shape
  num_emb, dh = row_embed.shape
  grid_size = int(math.sqrt(n)) + 1
  points = b * n
  assert points % (_NW * 128) == 0

  coords_flat = jnp.reshape(patch_coords, (2 * points,))
  out = _sc_lookup(coords_flat, row_embed, col_embed,
                   grid_size=grid_size, num_emb=num_emb, dh=dh)
  return jnp.reshape(out, (b, n, 2 * dh))


# speedup vs baseline: 1.8991x; 1.8991x over previous
import math
import functools

import jax
import jax.numpy as jnp
from jax import lax
from jax.experimental import pallas as pl
from jax.experimental.pallas import tpu as pltpu
from jax.experimental.pallas import tpu_sc as plsc

_NC = 2
_NS = 16
_NW = _NC * _NS
_L = 16


def _sc_lookup(coords_flat, row_embed, col_embed, *, grid_size, num_emb, dh):
  total = coords_flat.shape[0]
  points = total // 2
  ppw = points // _NW
  cpw = 2 * ppw
  chunk = 64
  n_chunks = ppw // chunk
  mchunk = 8192
  n_max_chunks = total // mchunk
  nv = min(grid_size + 1, num_emb)
  nvs = (nv + 7) // 8 * 8
  nvp = 3 * _L
  rpt = (nv + _NS - 1) // _NS

  mesh = plsc.VectorSubcoreMesh(
      core_axis_name="c", subcore_axis_name="s",
      num_cores=_NC, num_subcores=_NS)

  @functools.partial(
      pl.kernel,
      out_type=(
          jax.ShapeDtypeStruct((points, 2 * dh), jnp.float32),
          jax.ShapeDtypeStruct((nv * nvs, 2 * dh), jnp.float32),
      ),
      mesh=mesh,
      compiler_params=pltpu.CompilerParams(needs_layout_passes=False),
      scratch_types=[
          pltpu.VMEM((mchunk,), jnp.float32),
          pltpu.VMEM((cpw,), jnp.float32),
          pltpu.VMEM((ppw,), jnp.int32),
          pltpu.VMEM((nvp,), jnp.int32),
          pltpu.VMEM((nvp,), jnp.int32),
          pltpu.VMEM((chunk, 2 * dh), jnp.float32),
          pltpu.VMEM((chunk, 2 * dh), jnp.float32),
          pltpu.SemaphoreType.DMA,
          pltpu.SemaphoreType.DMA,
          pltpu.SemaphoreType.DMA,
          pltpu.SemaphoreType.DMA,
      ],
  )
  def body(coords_hbm, row_hbm, col_hbm, out_hbm, comb_hbm,
           mbuf, cbuf, cidx, ibr, ibc, kb0, kb1,
           gsem0, gsem1, wsem0, wsem1):
    cid = lax.axis_index("c")
    sid = lax.axis_index("s")
    wid = sid * _NC + cid
    kbufs = (kb0, kb1)
    gsems = (gsem0, gsem1)
    wsems = (wsem0, wsem1)

    for i in range(nvp // _L):
      ibc[pl.ds(i * _L, _L)] = lax.iota(jnp.int32, _L) + i * _L

    for rr in range(rpt):
      r = sid * rpt + rr

      @pl.when(r < nv)
      def _():
        for i in range(nvp // _L):
          ibr[pl.ds(i * _L, _L)] = jnp.full((_L,), r, dtype=jnp.int32)
        pltpu.async_copy(
            row_hbm.at[ibr], kb0.at[pl.ds(0, nvp), pl.ds(0, dh)], gsem0
        ).wait()
        pltpu.async_copy(
            col_hbm.at[ibc], kb0.at[pl.ds(0, nvp), pl.ds(dh, dh)], gsem0
        ).wait()
        pltpu.sync_copy(kb0.at[pl.ds(0, nvs)],
                        comb_hbm.at[pl.ds(r * nvs, nvs)])

    def max_step(j, accs):
      pltpu.sync_copy(coords_hbm.at[pl.ds(j * mchunk, mchunk)], mbuf)
      def red(i, accs):
        a0, a1, a2, a3 = accs
        base = i * (4 * _L)
        a0 = jnp.maximum(a0, mbuf[pl.ds(base, _L)])
        a1 = jnp.maximum(a1, mbuf[pl.ds(base + _L, _L)])
        a2 = jnp.maximum(a2, mbuf[pl.ds(base + 2 * _L, _L)])
        a3 = jnp.maximum(a3, mbuf[pl.ds(base + 3 * _L, _L)])
        return (a0, a1, a2, a3)
      return lax.fori_loop(0, mchunk // (4 * _L), red, accs)

    neg = jnp.full((_L,), -jnp.inf, dtype=jnp.float32)
    a0, a1, a2, a3 = lax.fori_loop(0, n_max_chunks, max_step,
                                   (neg, neg, neg, neg))
    acc = jnp.maximum(jnp.maximum(a0, a1), jnp.maximum(a2, a3))
    iota = lax.iota(jnp.int32, _L)
    for s in (1, 2, 4, 8):
      acc = jnp.maximum(acc, acc.at[iota ^ s].get(mode="promise_in_bounds"))
    max_coord = acc

    pltpu.sync_copy(coords_hbm.at[pl.ds(wid * cpw, cpw)], cbuf)
    gs = jnp.float32(grid_size)

    def idx_step(g, _):
      base = g * (2 * _L)
      xi = base + 2 * iota
      x = plsc.load_gather(cbuf, [xi])
      y = plsc.load_gather(cbuf, [xi + 1])
      r = jnp.clip((y / max_coord * gs).astype(jnp.int32), 0, nv - 1)
      c = jnp.clip((x / max_coord * gs).astype(jnp.int32), 0, nv - 1)
      cidx[pl.ds(g * _L, _L)] = r * nvs + c
      return 0

    lax.fori_loop(0, ppw // _L, idx_step, 0)

    plsc.subcore_barrier()

    def g_copies(k, b):
      p0 = k * chunk
      return (
          pltpu.make_async_copy(
              comb_hbm.at[cidx.at[pl.ds(p0, chunk)]], kbufs[b], gsems[b]),
      )

    def w_copies(k, b):
      o0 = wid * ppw + k * chunk
      return (
          pltpu.make_async_copy(
              kbufs[b], out_hbm.at[pl.ds(o0, chunk)], wsems[b]),
      )

    def issue(copies):
      for c in copies:
        c.start()

    def drain(copies):
      for c in copies:
        c.wait()

    issue(g_copies(0, 0))

    def pipe_step(j, _):
      k0 = 2 * j
      k1 = k0 + 1

      @pl.when(j > 0)
      def _():
        drain(w_copies(k1 - 2, 1))
      issue(g_copies(k1, 1))
      drain(g_copies(k0, 0))
      issue(w_copies(k0, 0))

      @pl.when(j < n_chunks // 2 - 1)
      def _():
        drain(w_copies(k0, 0))
        issue(g_copies(k0 + 2, 0))
      drain(g_copies(k1, 1))
      issue(w_copies(k1, 1))
      return 0

    lax.fori_loop(0, n_chunks // 2, pipe_step, 0)
    drain(w_copies(n_chunks - 2, 0))
    drain(w_copies(n_chunks - 1, 1))

  return body(coords_flat, row_embed, col_embed)


def kernel(patch_coords, row_embed, col_embed):
  b, n, _ = patch_coords.shape
  num_emb, dh = row_embed.shape
  grid_size = int(math.sqrt(n)) + 1
  points = b * n
  assert points % (_NW * 128) == 0

  coords_flat = jnp.reshape(patch_coords, (2 * points,))
  out, _unused_comb = _sc_lookup(coords_flat, row_embed, col_embed,
                                 grid_size=grid_size, num_emb=num_emb, dh=dh)
  return jnp.reshape(out, (b, n, 2 * dh))


# speedup vs baseline: 1.9466x; 1.0250x over previous
import math
import functools

import jax
import jax.numpy as jnp
from jax import lax
from jax.experimental import pallas as pl
from jax.experimental.pallas import tpu as pltpu
from jax.experimental.pallas import tpu_sc as plsc

_NC = 2
_NS = 16
_NW = _NC * _NS
_L = 16


def _sc_lookup(coords_flat, row_embed, col_embed, *, grid_size, num_emb, dh):
  total = coords_flat.shape[0]
  points = total // 2
  ppw = points // _NW
  cpw = 2 * ppw
  chunk = 32
  nb = 4
  n_chunks = ppw // chunk
  mchunk = 8192
  n_max_chunks = total // mchunk
  nv = min(grid_size + 1, num_emb)
  nvs = (nv + 7) // 8 * 8
  nvp = 3 * _L
  rpt = (nv + _NS - 1) // _NS

  mesh = plsc.VectorSubcoreMesh(
      core_axis_name="c", subcore_axis_name="s",
      num_cores=_NC, num_subcores=_NS)

  @functools.partial(
      pl.kernel,
      out_type=(
          jax.ShapeDtypeStruct((points, 2 * dh), jnp.float32),
          jax.ShapeDtypeStruct((nv * nvs, 2 * dh), jnp.float32),
      ),
      mesh=mesh,
      compiler_params=pltpu.CompilerParams(needs_layout_passes=False),
      scratch_types=[
          pltpu.VMEM((mchunk,), jnp.float32),
          pltpu.VMEM((cpw,), jnp.float32),
          pltpu.VMEM((ppw,), jnp.int32),
          pltpu.VMEM((nvp,), jnp.int32),
          pltpu.VMEM((nvp,), jnp.int32),
      ] + [pltpu.VMEM((chunk, 2 * dh), jnp.float32)] * nb
        + [pltpu.SemaphoreType.DMA] * (2 * nb),
  )
  def body(coords_hbm, row_hbm, col_hbm, out_hbm, comb_hbm,
           mbuf, cbuf, cidx, ibr, ibc, *bufs_sems):
    kbufs = bufs_sems[:nb]
    gsems = bufs_sems[nb:2 * nb]
    wsems = bufs_sems[2 * nb:3 * nb]
    cid = lax.axis_index("c")
    sid = lax.axis_index("s")
    wid = sid * _NC + cid

    for i in range(nvp // _L):
      ibc[pl.ds(i * _L, _L)] = lax.iota(jnp.int32, _L) + i * _L

    lo = chunk
    hi = nvs - chunk
    for rr in range(rpt):
      r = sid * rpt + rr

      @pl.when(r < nv)
      def _():
        for i in range(nvp // _L):
          ibr[pl.ds(i * _L, _L)] = jnp.full((_L,), r, dtype=jnp.int32)
        cps = (
            pltpu.make_async_copy(
                row_hbm.at[ibr.at[pl.ds(0, lo)]],
                kbufs[0].at[:, pl.ds(0, dh)], gsems[0]),
            pltpu.make_async_copy(
                col_hbm.at[ibc.at[pl.ds(0, lo)]],
                kbufs[0].at[:, pl.ds(dh, dh)], gsems[0]),
            pltpu.make_async_copy(
                row_hbm.at[ibr.at[pl.ds(lo, hi)]],
                kbufs[1].at[pl.ds(0, hi), pl.ds(0, dh)], gsems[1]),
            pltpu.make_async_copy(
                col_hbm.at[ibc.at[pl.ds(lo, hi)]],
                kbufs[1].at[pl.ds(0, hi), pl.ds(dh, dh)], gsems[1]),
        )
        for cp in cps:
          cp.start()
        for cp in cps:
          cp.wait()
        pltpu.sync_copy(kbufs[0], comb_hbm.at[pl.ds(r * nvs, lo)])
        pltpu.sync_copy(kbufs[1].at[pl.ds(0, hi)],
                        comb_hbm.at[pl.ds(r * nvs + lo, hi)])

    def max_step(j, accs):
      pltpu.sync_copy(coords_hbm.at[pl.ds(j * mchunk, mchunk)], mbuf)
      def red(i, accs):
        a0, a1, a2, a3 = accs
        base = i * (4 * _L)
        a0 = jnp.maximum(a0, mbuf[pl.ds(base, _L)])
        a1 = jnp.maximum(a1, mbuf[pl.ds(base + _L, _L)])
        a2 = jnp.maximum(a2, mbuf[pl.ds(base + 2 * _L, _L)])
        a3 = jnp.maximum(a3, mbuf[pl.ds(base + 3 * _L, _L)])
        return (a0, a1, a2, a3)
      return lax.fori_loop(0, mchunk // (4 * _L), red, accs)

    neg = jnp.full((_L,), -jnp.inf, dtype=jnp.float32)
    a0, a1, a2, a3 = lax.fori_loop(0, n_max_chunks, max_step,
                                   (neg, neg, neg, neg))
    acc = jnp.maximum(jnp.maximum(a0, a1), jnp.maximum(a2, a3))
    iota = lax.iota(jnp.int32, _L)
    for s in (1, 2, 4, 8):
      acc = jnp.maximum(acc, acc.at[iota ^ s].get(mode="promise_in_bounds"))
    max_coord = acc

    pltpu.sync_copy(coords_hbm.at[pl.ds(wid * cpw, cpw)], cbuf)
    gs = jnp.float32(grid_size)

    def idx_step(g, _):
      base = g * (2 * _L)
      xi = base + 2 * iota
      x = plsc.load_gather(cbuf, [xi])
      y = plsc.load_gather(cbuf, [xi + 1])
      r = jnp.clip((y / max_coord * gs).astype(jnp.int32), 0, nv - 1)
      c = jnp.clip((x / max_coord * gs).astype(jnp.int32), 0, nv - 1)
      cidx[pl.ds(g * _L, _L)] = r * nvs + c
      return 0

    lax.fori_loop(0, ppw // _L, idx_step, 0)

    plsc.subcore_barrier()

    def g_copies(k, b):
      p0 = k * chunk
      return (
          pltpu.make_async_copy(
              comb_hbm.at[cidx.at[pl.ds(p0, chunk)]], kbufs[b], gsems[b]),
      )

    def w_copies(k, b):
      o0 = wid * ppw + k * chunk
      return (
          pltpu.make_async_copy(
              kbufs[b], out_hbm.at[pl.ds(o0, chunk)], wsems[b]),
      )

    def issue(copies):
      for c in copies:
        c.start()

    def drain(copies):
      for c in copies:
        c.wait()

    for b in range(nb):
      issue(g_copies(b, b))

    n_super = n_chunks // nb

    def pipe_step(jj, _):
      for b in range(nb):
        k = jj * nb + b
        drain(g_copies(k, b))
        issue(w_copies(k, b))

        @pl.when(jj < n_super - 1)
        def _():
          drain(w_copies(k, b))
          issue(g_copies(k + nb, b))
      return 0

    lax.fori_loop(0, n_super, pipe_step, 0)
    for b in range(nb):
      drain(w_copies(n_chunks - nb + b, b))

  return body(coords_flat, row_embed, col_embed)


def kernel(patch_coords, row_embed, col_embed):
  b, n, _ = patch_coords.shape
  num_emb, dh = row_embed.shape
  grid_size = int(math.sqrt(n)) + 1
  points = b * n
  assert points % (_NW * 128) == 0

  coords_flat = jnp.reshape(patch_coords, (2 * points,))
  out, _unused_comb = _sc_lookup(coords_flat, row_embed, col_embed,
                                 grid_size=grid_size, num_emb=num_emb, dh=dh)
  return jnp.reshape(out, (b, n, 2 * dh))
